# Initial kernel scaffold; baseline (speedup 1.0000x reference)
import jax
import jax.numpy as jnp
from jax.experimental import pallas as pl


def kernel(x_enc, x_mark_enc, x_dec, x_mark_dec, start_W, start_b, w_gate, W1, b1, W2, b2, proj_W, proj_b):
    raise NotImplementedError("write your pallas kernel here")


# speedup vs baseline: 5.0623x; 5.0623x over previous
import jax
import jax.numpy as jnp
from jax.experimental import pallas as pl
from jax.experimental.pallas import tpu as pltpu

_B, _L, _N, _D, _F, _E, _K, _P, _LAYERS = 32, 96, 32, 16, 64, 4, 2, 96, 2
_EF = _E * _F


def _model_kernel(x_ref, startW_ref, startb_ref, wg_ref, W1_ref, b1_ref,
                  W2_ref, b2_ref, sel_ref, projW_ref, projb_ref,
                  dec_ref, bal_ref, imp_ref, load_ref, M_ref):
    b = pl.program_id(0)
    f32 = jnp.float32

    xt = x_ref[0]
    out = xt * startW_ref[:] + startb_ref[:]

    imps, loads = [], []
    for l in range(_LAYERS):
        logits = jnp.dot(out, wg_ref[l], preferred_element_type=f32)
        iota_e = jax.lax.broadcasted_iota(jnp.int32, logits.shape, 1)
        v1 = jnp.max(logits, axis=1, keepdims=True)
        i1 = jnp.min(jnp.where(logits == v1, iota_e, _E), axis=1, keepdims=True)
        oh1 = (iota_e == i1).astype(f32)
        masked = jnp.where(iota_e == i1, -jnp.inf, logits)
        v2 = jnp.max(masked, axis=1, keepdims=True)
        i2 = jnp.min(jnp.where(masked == v2, iota_e, _E), axis=1, keepdims=True)
        oh2 = (iota_e == i2).astype(f32)
        e2 = jnp.exp(v2 - v1)
        denom = 1.0 + e2
        gates = (1.0 / denom) * oh1 + (e2 / denom) * oh2
        imps.append(jnp.sum(gates, axis=0, keepdims=True))
        loads.append(jnp.sum((gates > 0).astype(f32), axis=0, keepdims=True))

        h = jax.nn.gelu(jnp.dot(out, W1_ref[l], preferred_element_type=f32)
                        + b1_ref[l])
        ge = jnp.dot(gates, sel_ref[:], preferred_element_type=f32)
        y = (jnp.dot(h * ge, W2_ref[l], preferred_element_type=f32)
             + jnp.dot(gates, b2_ref[l], preferred_element_type=f32))
        out = out + y

    for l in range(_L):
        M_ref[:, l * _D:(l + 1) * _D] = out[l * _N:(l + 1) * _N, :]
    z = (jnp.dot(M_ref[:], projW_ref[:],
                 preferred_element_type=f32) + projb_ref[:])
    dec_ref[0] = z.T

    @pl.when(b == 0)
    def _init():
        for l in range(_LAYERS):
            imp_ref[l] = imps[l]
            load_ref[l] = loads[l]

    @pl.when(b != 0)
    def _acc():
        for l in range(_LAYERS):
            imp_ref[l] += imps[l]
            load_ref[l] += loads[l]

    @pl.when(b == _B - 1)
    def _finalize():
        bal = jnp.zeros((1, 1), dtype=f32)
        for l in range(_LAYERS):
            for ref in (imp_ref, load_ref):
                v = ref[l]
                m = jnp.sum(v, keepdims=True) / _E
                var = jnp.sum((v - m) ** 2, keepdims=True) / (_E - 1)
                bal = bal + var / (m * m + 1e-10)
        bal_ref[:] = bal


def kernel(x_enc, x_mark_enc, x_dec, x_mark_dec, start_W, start_b, w_gate,
           W1, b1, W2, b2, proj_W, proj_b):
    f32 = jnp.float32
    W1c = jnp.transpose(W1, (0, 2, 1, 3)).reshape(_LAYERS, _D, _EF)
    b1c = b1.reshape(_LAYERS, 1, _EF)
    W2c = W2.reshape(_LAYERS, _EF, _D)
    startW = start_W.reshape(1, _D)
    startb = start_b.reshape(1, _D)
    projb = proj_b.reshape(1, _P)
    sel = jnp.repeat(jnp.eye(_E, dtype=f32), _F, axis=1)
    xp = x_enc.reshape(_B, _L * _N, 1)

    full = lambda shape: pl.BlockSpec(shape, lambda b: (0,) * len(shape))
    dec, bal = pl.pallas_call(
        _model_kernel,
        grid=(_B,),
        in_specs=[
            pl.BlockSpec((1, _N * _L, 1), lambda b: (b, 0, 0)),
            full((1, _D)),
            full((1, _D)),
            full((_LAYERS, _D, _E)),
            full((_LAYERS, _D, _EF)),
            full((_LAYERS, 1, _EF)),
            full((_LAYERS, _EF, _D)),
            full((_LAYERS, _E, _D)),
            full((_E, _EF)),
            full((_L * _D, _P)),
            full((1, _P)),
        ],
        out_specs=[
            pl.BlockSpec((1, _P, _N), lambda b: (b, 0, 0)),
            pl.BlockSpec((1, 1), lambda b: (0, 0)),
        ],
        out_shape=[
            jax.ShapeDtypeStruct((_B, _P, _N), f32),
            jax.ShapeDtypeStruct((1, 1), f32),
        ],
        scratch_shapes=[
            pltpu.VMEM((_LAYERS, 1, _E), f32),
            pltpu.VMEM((_LAYERS, 1, _E), f32),
            pltpu.VMEM((_N, _L * _D), f32),
        ],
        compiler_params=pltpu.CompilerParams(
            dimension_semantics=("arbitrary",),
        ),
    )(xp, startW, startb, w_gate, W1c, b1c, W2c, b2, sel, proj_W, projb)
    return dec, bal[0, 0]


# speedup vs baseline: 14.8741x; 2.9382x over previous
import jax
import jax.numpy as jnp
from jax.experimental import pallas as pl
from jax.experimental.pallas import tpu as pltpu

_B, _L, _N, _D, _F, _E, _K, _P, _LAYERS = 32, 96, 32, 16, 64, 4, 2, 96, 2
_EF = _E * _F
_T = _L * _N


def _top2(lgT):
    f32 = jnp.float32

    def first_max_onehot(x):
        v = jnp.max(x, axis=0, keepdims=True)
        rows = []
        seen = jnp.zeros_like(v)
        for e in range(_E):
            eq = (x[e:e + 1] == v).astype(f32)
            rows.append(eq * (1.0 - seen))
            seen = jnp.maximum(seen, eq)
        return jnp.concatenate(rows, axis=0), v

    oh1, v1 = first_max_onehot(lgT)
    masked = jnp.where(oh1 > 0, -jnp.inf, lgT)
    oh2, v2 = first_max_onehot(masked)
    e2 = jnp.exp(v2 - v1)
    den = 1.0 + e2
    return oh1 * (1.0 / den) + oh2 * (e2 / den)


def _model_kernel(x_ref, startW_ref, startb_ref, wg_ref, W1_ref, b1_ref,
                  W2_ref, b2_ref, projW_ref, projb_ref,
                  dec_ref, bal_ref, imp_ref, load_ref, M_ref):
    b = pl.program_id(0)
    f32 = jnp.float32

    xt = x_ref[0]
    outT = startW_ref[:] * xt + startb_ref[:]

    imps, loads = [], []
    for l in range(_LAYERS):
        lgT = jnp.dot(wg_ref[l], outT, preferred_element_type=f32)
        gatesT = _top2(lgT)
        imps.append(jnp.sum(gatesT, axis=1, keepdims=True))
        loads.append(jnp.sum((gatesT > 0).astype(f32), axis=1, keepdims=True))

        hT = jax.nn.gelu(jnp.dot(W1_ref[l], outT, preferred_element_type=f32)
                         + b1_ref[l])
        hg = jnp.concatenate(
            [hT[e * _F:(e + 1) * _F] * gatesT[e:e + 1] for e in range(_E)],
            axis=0)
        yT = (jnp.dot(W2_ref[l], hg, preferred_element_type=f32)
              + jnp.dot(b2_ref[l], gatesT, preferred_element_type=f32))
        outT = outT + yT

    for l in range(_L):
        M_ref[l * _D:(l + 1) * _D, :] = outT[:, l * _N:(l + 1) * _N]
    zT = (jnp.dot(projW_ref[:], M_ref[:], preferred_element_type=f32)
          + projb_ref[:])
    dec_ref[0] = zT

    @pl.when(b == 0)
    def _init():
        for l in range(_LAYERS):
            imp_ref[l] = imps[l]
            load_ref[l] = loads[l]

    @pl.when(b != 0)
    def _acc():
        for l in range(_LAYERS):
            imp_ref[l] += imps[l]
            load_ref[l] += loads[l]

    @pl.when(b == _B - 1)
    def _finalize():
        bal = jnp.zeros((1, 1), dtype=f32)
        for l in range(_LAYERS):
            for ref in (imp_ref, load_ref):
                v = ref[l]
                m = jnp.sum(v, keepdims=True) / _E
                var = jnp.sum((v - m) ** 2, keepdims=True) / (_E - 1)
                bal = bal + var / (m * m + 1e-10)
        bal_ref[:] = bal


def kernel(x_enc, x_mark_enc, x_dec, x_mark_dec, start_W, start_b, w_gate,
           W1, b1, W2, b2, proj_W, proj_b):
    f32 = jnp.float32
    wgT = jnp.transpose(w_gate, (0, 2, 1))
    W1T = jnp.transpose(W1, (0, 1, 3, 2)).reshape(_LAYERS, _EF, _D)
    b1T = b1.reshape(_LAYERS, _EF, 1)
    W2T = jnp.transpose(W2, (0, 3, 1, 2)).reshape(_LAYERS, _D, _EF)
    b2T = jnp.transpose(b2, (0, 2, 1))
    startWT = start_W.reshape(_D, 1)
    startbT = start_b.reshape(_D, 1)
    projWT = jnp.transpose(proj_W, (1, 0))
    projbT = proj_b.reshape(_P, 1)
    xp = x_enc.reshape(_B, 1, _T)

    full = lambda shape: pl.BlockSpec(shape, lambda b: (0,) * len(shape))
    dec, bal = pl.pallas_call(
        _model_kernel,
        grid=(_B,),
        in_specs=[
            pl.BlockSpec((1, 1, _T), lambda b: (b, 0, 0)),
            full((_D, 1)),
            full((_D, 1)),
            full((_LAYERS, _E, _D)),
            full((_LAYERS, _EF, _D)),
            full((_LAYERS, _EF, 1)),
            full((_LAYERS, _D, _EF)),
            full((_LAYERS, _D, _E)),
            full((_P, _L * _D)),
            full((_P, 1)),
        ],
        out_specs=[
            pl.BlockSpec((1, _P, _N), lambda b: (b, 0, 0)),
            pl.BlockSpec((1, 1), lambda b: (0, 0)),
        ],
        out_shape=[
            jax.ShapeDtypeStruct((_B, _P, _N), f32),
            jax.ShapeDtypeStruct((1, 1), f32),
        ],
        scratch_shapes=[
            pltpu.VMEM((_LAYERS, _E, 1), f32),
            pltpu.VMEM((_LAYERS, _E, 1), f32),
            pltpu.VMEM((_L * _D, _N), f32),
        ],
        compiler_params=pltpu.CompilerParams(
            dimension_semantics=("arbitrary",),
        ),
    )(xp, startWT, startbT, wgT, W1T, b1T, W2T, b2T, projWT, projbT)
    return dec, bal[0, 0]


# speedup vs baseline: 16.6520x; 1.1195x over previous
import jax
import jax.numpy as jnp
from jax.experimental import pallas as pl
from jax.experimental.pallas import tpu as pltpu

_B, _L, _N, _D, _F, _E, _K, _P, _LAYERS = 32, 96, 32, 16, 64, 4, 2, 96, 2
_EF = _E * _F
_BLK = 4
_T = _BLK * _L * _N
_TB = _L * _N
_C1 = 0.7978845608028654
_C2 = _C1 * 0.044715


def _top2(lgT):
    f32 = jnp.float32

    def first_max_onehot(x):
        v = jnp.max(x, axis=0, keepdims=True)
        rows = []
        seen = jnp.zeros_like(v)
        for e in range(_E):
            eq = (x[e:e + 1] == v).astype(f32)
            rows.append(eq * (1.0 - seen))
            seen = jnp.maximum(seen, eq)
        return jnp.concatenate(rows, axis=0), v

    oh1, v1 = first_max_onehot(lgT)
    masked = jnp.where(oh1 > 0, -jnp.inf, lgT)
    oh2, v2 = first_max_onehot(masked)
    e2 = jnp.exp(v2 - v1)
    den = 1.0 + e2
    return oh1 * (1.0 / den) + oh2 * (e2 / den)


def _gated_gelu(x, halfg):
    u = x * x
    t = jnp.tanh(x * (_C1 + _C2 * u))
    s = halfg * x
    return s + s * t


def _model_kernel(x_ref, startW_ref, startb_ref, wg_ref, W1_ref, b1_ref,
                  W2_ref, b2_ref, projW_ref, projb_ref,
                  dec_ref, bal_ref, imp_ref, load_ref, M_ref):
    b = pl.program_id(0)
    f32 = jnp.float32

    xt = x_ref[0]
    outT = startW_ref[:] * xt + startb_ref[:]

    imps, loads = [], []
    for l in range(_LAYERS):
        lgT = jnp.dot(wg_ref[l], outT, preferred_element_type=f32)
        gatesT = _top2(lgT)
        imps.append(jnp.sum(gatesT, axis=1, keepdims=True))
        loads.append(jnp.sum((gatesT > 0).astype(f32), axis=1, keepdims=True))

        hT = jax.nn.gelu(jnp.dot(W1_ref[l], outT, preferred_element_type=f32)
                         + b1_ref[l])
        yT = None
        for e in range(_E):
            oeT = (jnp.dot(W2_ref[l][:, e * _F:(e + 1) * _F],
                           hT[e * _F:(e + 1) * _F],
                           preferred_element_type=f32)
                   + b2_ref[l][:, e:e + 1])
            term = gatesT[e:e + 1] * oeT
            yT = term if yT is None else yT + term
        outT = outT + yT

    for k in range(_BLK):
        for l in range(_L):
            M_ref[k, l * _D:(l + 1) * _D, :] = (
                outT[:, k * _TB + l * _N:k * _TB + (l + 1) * _N])
    for k in range(_BLK):
        dec_ref[k] = (jnp.dot(projW_ref[:], M_ref[k], preferred_element_type=f32)
                      + projb_ref[:])

    @pl.when(b == 0)
    def _init():
        for l in range(_LAYERS):
            imp_ref[l] = imps[l]
            load_ref[l] = loads[l]

    @pl.when(b != 0)
    def _acc():
        for l in range(_LAYERS):
            imp_ref[l] += imps[l]
            load_ref[l] += loads[l]

    @pl.when(b == _B // _BLK - 1)
    def _finalize():
        bal = jnp.zeros((1, 1), dtype=f32)
        for l in range(_LAYERS):
            for ref in (imp_ref, load_ref):
                v = ref[l]
                m = jnp.sum(v, keepdims=True) / _E
                var = jnp.sum((v - m) ** 2, keepdims=True) / (_E - 1)
                bal = bal + var / (m * m + 1e-10)
        bal_ref[:] = bal


def kernel(x_enc, x_mark_enc, x_dec, x_mark_dec, start_W, start_b, w_gate,
           W1, b1, W2, b2, proj_W, proj_b):
    f32 = jnp.float32
    wgT = jnp.transpose(w_gate, (0, 2, 1))
    W1T = jnp.transpose(W1, (0, 1, 3, 2)).reshape(_LAYERS, _EF, _D)
    b1T = b1.reshape(_LAYERS, _EF, 1)
    W2T = jnp.transpose(W2, (0, 3, 1, 2)).reshape(_LAYERS, _D, _EF)
    b2T = jnp.transpose(b2, (0, 2, 1))
    startWT = start_W.reshape(_D, 1)
    startbT = start_b.reshape(_D, 1)
    projWT = jnp.transpose(proj_W, (1, 0))
    projbT = proj_b.reshape(_P, 1)
    xp = x_enc.reshape(_B // _BLK, 1, _T)

    full = lambda shape: pl.BlockSpec(shape, lambda b: (0,) * len(shape))
    dec, bal = pl.pallas_call(
        _model_kernel,
        grid=(_B // _BLK,),
        in_specs=[
            pl.BlockSpec((1, 1, _T), lambda b: (b, 0, 0)),
            full((_D, 1)),
            full((_D, 1)),
            full((_LAYERS, _E, _D)),
            full((_LAYERS, _EF, _D)),
            full((_LAYERS, _EF, 1)),
            full((_LAYERS, _D, _EF)),
            full((_LAYERS, _D, _E)),
            full((_P, _L * _D)),
            full((_P, 1)),
        ],
        out_specs=[
            pl.BlockSpec((_BLK, _P, _N), lambda b: (b, 0, 0)),
            pl.BlockSpec((1, 1), lambda b: (0, 0)),
        ],
        out_shape=[
            jax.ShapeDtypeStruct((_B, _P, _N), f32),
            jax.ShapeDtypeStruct((1, 1), f32),
        ],
        scratch_shapes=[
            pltpu.VMEM((_LAYERS, _E, 1), f32),
            pltpu.VMEM((_LAYERS, _E, 1), f32),
            pltpu.VMEM((_BLK, _L * _D, _N), f32),
        ],
        compiler_params=pltpu.CompilerParams(
            dimension_semantics=("arbitrary",),
        ),
    )(xp, startWT, startbT, wgT, W1T, b1T, W2T, b2T, projWT, projbT)
    return dec, bal[0, 0]
